# baseline (speedup 1.0000x reference)
import jax
import jax.numpy as jnp
from jax.experimental import pallas as pl
from jax.experimental.pallas import tpu as pltpu

_NUM_PCS = 5
_PCS_RANGE = 2.0
_SPACING = 2.0 * _PCS_RANGE / _NUM_PCS
_INV_SPACING = 1.0 / _SPACING

_BM1 = 128
_BM2 = 40


def _plf_compute(p, c):
    t = p * _INV_SPACING + (_NUM_PCS / 2)
    left = jnp.clip(jnp.floor(t), 0.0, _NUM_PCS - 1.0)
    w = t - left
    m = [left == k for k in range(_NUM_PCS - 1)]
    cl = c[_NUM_PCS - 1]
    cr = c[_NUM_PCS]
    for k in range(_NUM_PCS - 2, -1, -1):
        cl = jnp.where(m[k], c[k], cl)
        cr = jnp.where(m[k], c[k + 1], cr)
    return (1.0 - w) * cl + w * cr


def kernel(param1, param2, ctrl1, ctrl2):
    r1, cols = param1.shape
    r2, _ = param2.shape
    n1 = r1 // _BM1
    n2 = r2 // _BM2
    ctrl1_t = jnp.transpose(ctrl1, (2, 0, 1))
    ctrl2_t = jnp.transpose(ctrl2, (2, 0, 1))

    def fused_kernel(p1_ref, c1_ref, p2_ref, c2_ref, o1_ref, o2_ref):
        i = pl.program_id(0)

        @pl.when(i < n1)
        def _():
            o1_ref[...] = _plf_compute(
                p1_ref[...], [c1_ref[k] for k in range(_NUM_PCS + 1)]
            )

        @pl.when(i >= n1)
        def _():
            o2_ref[...] = _plf_compute(
                p2_ref[...], [c2_ref[k] for k in range(_NUM_PCS + 1)]
            )

    def i1(i):
        return (jnp.minimum(i, n1 - 1), 0)

    def ci1(i):
        return (0, jnp.minimum(i, n1 - 1), 0)

    def i2(i):
        return (jnp.clip(i - n1, 0, n2 - 1), 0)

    def ci2(i):
        return (0, jnp.clip(i - n1, 0, n2 - 1), 0)

    o1, o2 = pl.pallas_call(
        fused_kernel,
        grid=(n1 + n2,),
        in_specs=[
            pl.BlockSpec((_BM1, cols), i1),
            pl.BlockSpec((_NUM_PCS + 1, _BM1, cols), ci1),
            pl.BlockSpec((_BM2, cols), i2),
            pl.BlockSpec((_NUM_PCS + 1, _BM2, cols), ci2),
        ],
        out_specs=[
            pl.BlockSpec((_BM1, cols), i1),
            pl.BlockSpec((_BM2, cols), i2),
        ],
        out_shape=[
            jax.ShapeDtypeStruct((r1, cols), param1.dtype),
            jax.ShapeDtypeStruct((r2, cols), param2.dtype),
        ],
        compiler_params=pltpu.CompilerParams(
            dimension_semantics=("parallel",),
            vmem_limit_bytes=100 * 1024 * 1024,
        ),
    )(param1, ctrl1_t, param2, ctrl2_t)
    return (o1, o2)


# speedup vs baseline: 1.0072x; 1.0072x over previous
import jax
import jax.numpy as jnp
from jax.experimental import pallas as pl
from jax.experimental.pallas import tpu as pltpu

_NUM_PCS = 5
_PCS_RANGE = 2.0
_SPACING = 2.0 * _PCS_RANGE / _NUM_PCS
_INV_SPACING = 1.0 / _SPACING


def _plf_kernel(p_ref, c_ref, o_ref):
    p = p_ref[...]
    t = p * _INV_SPACING + (_NUM_PCS / 2)
    left = jnp.clip(jnp.floor(t), 0.0, _NUM_PCS - 1.0)
    w = t - left
    c = [c_ref[k] for k in range(_NUM_PCS + 1)]
    m = [left == k for k in range(_NUM_PCS - 1)]
    cl = c[_NUM_PCS - 1]
    cr = c[_NUM_PCS]
    for k in range(_NUM_PCS - 2, -1, -1):
        cl = jnp.where(m[k], c[k], cl)
        cr = jnp.where(m[k], c[k + 1], cr)
    o_ref[...] = (1.0 - w) * cl + w * cr


def _plf_call(param, ctrl, block_rows, block_cols=None, col_parallel=False):
    rows, cols = param.shape
    bc = cols if block_cols is None else block_cols
    ctrl_t = jnp.transpose(ctrl, (2, 0, 1))
    if col_parallel:
        grid = (cols // bc, rows // block_rows)
        pmap = lambda j, i: (i, j)
        cmap = lambda j, i: (0, i, j)
    else:
        grid = (rows // block_rows, cols // bc)
        pmap = lambda i, j: (i, j)
        cmap = lambda i, j: (0, i, j)
    return pl.pallas_call(
        _plf_kernel,
        grid=grid,
        in_specs=[
            pl.BlockSpec((block_rows, bc), pmap),
            pl.BlockSpec((_NUM_PCS + 1, block_rows, bc), cmap),
        ],
        out_specs=pl.BlockSpec((block_rows, bc), pmap),
        out_shape=jax.ShapeDtypeStruct((rows, cols), param.dtype),
        compiler_params=pltpu.CompilerParams(
            dimension_semantics=("parallel", "arbitrary"),
            vmem_limit_bytes=100 * 1024 * 1024,
        ),
    )(param, ctrl_t)


def kernel(param1, param2, ctrl1, ctrl2):
    return (
        _plf_call(param1, ctrl1, 128),
        _plf_call(param2, ctrl2, 200, 2048, col_parallel=True),
    )
